# baseline (speedup 1.0000x reference)
import dataclasses
import functools

import jax
import jax.numpy as jnp
from jax import lax
from jax.experimental import pallas as pl
from jax.experimental.pallas import tpu as pltpu
from jax.experimental.pallas import tpu_sc as plsc

DIM = 32
R = 8
LANES = 16
NTEC = 32
TOK_SPLITS = ((0, 64), (64, 64), (128, 64), (192, 8))
TC_BC = 512


def _tc_relayout_table(table):
    v = table.shape[0]
    nb = pl.cdiv(v, TC_BC)

    def body(x_ref, o_ref):
        x = x_ref[...]
        parts = [x[:, 128 * a:128 * (a + 1)].T for a in range(TC_BC // 128)]
        o_ref[...] = jnp.concatenate(parts, axis=1)

    return pl.pallas_call(
        body,
        grid=(nb,),
        in_specs=[pl.BlockSpec((DIM, TC_BC), lambda i: (0, i))],
        out_specs=pl.BlockSpec((128, 128), lambda i: (i, 0)),
        out_shape=jax.ShapeDtypeStruct((nb * 128, 128), jnp.float32),
    )(table.T)


def _zero_padding_rows(idx_row, out_row):
    zeros = jnp.zeros((LANES,), jnp.float32)
    for off in list(range(0, 192, LANES)) + [200 - LANES]:
        v = idx_row[pl.ds(off, LANES)]
        is_pad = v == 0

        @pl.when(jnp.any(is_pad))
        def _():
            rows = (jnp.arange(LANES, dtype=jnp.int32) + off) * 2

            @pl.loop(0, 16)
            def _(c):
                cols = jnp.full((LANES,), 0, jnp.int32) + c
                plsc.store_scatter(out_row, [rows, cols], zeros, mask=is_pad)
                plsc.store_scatter(out_row, [rows + 1, cols], zeros,
                                   mask=is_pad)


def _make_kernel(n_rows, n_cols):
    mesh = plsc.VectorSubcoreMesh(core_axis_name="c", subcore_axis_name="s")
    out_sds = jax.ShapeDtypeStruct((n_rows, 2 * n_cols, 16), jnp.float32)
    rows_per_tec = n_rows // NTEC
    n_ch = rows_per_tec // R

    cp = pltpu.CompilerParams()
    fields = pltpu.CompilerParams.__dataclass_fields__
    if "needs_layout_passes" in fields:
        cp = dataclasses.replace(cp, needs_layout_passes=False)
    if "use_tc_tiling_on_sc" in fields:
        cp = dataclasses.replace(cp, use_tc_tiling_on_sc=False)

    @functools.partial(
        pl.kernel,
        out_type=(out_sds, out_sds),
        mesh=mesh,
        compiler_params=cp,
        scratch_types=[
            pltpu.VMEM((R, n_cols), jnp.int32),
            pltpu.VMEM((R, n_cols), jnp.int32),
            pltpu.VMEM((R, 2 * n_cols), jnp.int32),
            pltpu.VMEM((R, 2 * n_cols), jnp.int32),
            pltpu.VMEM((R, 2 * n_cols, 16), jnp.float32),
            pltpu.VMEM((R, 2 * n_cols, 16), jnp.float32),
            pltpu.SemaphoreType.DMA,
            pltpu.SemaphoreType.DMA,
            pltpu.SemaphoreType.DMA,
            pltpu.SemaphoreType.DMA,
            pltpu.SemaphoreType.DMA,
            pltpu.SemaphoreType.DMA,
        ],
    )
    def emb_kernel(src_table_hbm, tgt_table_hbm, src_tok_hbm, tgt_tok_hbm,
                   src_out_hbm, tgt_out_hbm,
                   idx0, idx1, didx0, didx1, outb0, outb1,
                   si0, si1, sg0, sg1, so0, so1):
        wid = lax.axis_index("s") * 2 + lax.axis_index("c")
        base_row = wid * rows_per_tec
        idxb = (idx0, idx1)
        didxb = (didx0, didx1)
        outb = (outb0, outb1)
        si = (si0, si1)
        sg = (sg0, sg1)
        so = (so0, so1)

        def run(tbl, tok, out):
            def row0(c):
                return base_row + c * R

            def idx_req(c, b):
                pltpu.async_copy(tok.at[pl.ds(row0(c), R)], idxb[b], si[b])

            def idx_wait(b):
                pltpu.make_async_copy(
                    tok.at[pl.ds(0, R)], idxb[b], si[b]).wait()

            def expand(b):
                even = jnp.arange(LANES, dtype=jnp.int32) * 2

                @pl.loop(0, R)
                def _(j):
                    drow = didxb[b].at[j]
                    irow = idxb[b].at[j]
                    for off in list(range(0, 192, LANES)) + [200 - LANES]:
                        r = irow[pl.ds(off, LANES)]
                        g = ((r >> 9) * 1024 + (r & 127) * 8
                             + ((r >> 7) & 3) * 2)
                        lanes = even + (2 * off)
                        plsc.store_scatter(drow, [lanes], g)
                        plsc.store_scatter(drow, [lanes + 1], g + 1)

            def fire(b):
                @pl.loop(0, R)
                def _(j):
                    for off, w in TOK_SPLITS:
                        pltpu.async_copy(
                            tbl.at[didxb[b].at[j, pl.ds(2 * off, 2 * w)]],
                            outb[b].at[j, pl.ds(2 * off, 2 * w)], sg[b])

            def drain(b):
                @pl.loop(0, R)
                def _(j):
                    for off, w in TOK_SPLITS:
                        pltpu.make_async_copy(
                            tbl.at[didxb[b].at[j, pl.ds(2 * off, 2 * w)]],
                            outb[b].at[j, pl.ds(2 * off, 2 * w)], sg[b]).wait()

            def wb_start(c, b):
                pltpu.async_copy(outb[b], out.at[pl.ds(row0(c), R)], so[b])

            def wb_wait(b):
                pltpu.make_async_copy(
                    outb[b], out.at[pl.ds(0, R)], so[b]).wait()

            idx_req(0, 0)
            idx_req(1, 1)
            idx_wait(0)
            expand(0)
            fire(0)

            @pl.loop(0, n_ch // 2)
            def _(k):
                for half in (0, 1):
                    c = 2 * k + half
                    b = half
                    nb = 1 - half
                    @pl.when(c + 1 < n_ch)
                    def _():
                        idx_wait(nb)
                        expand(nb)

                    @pl.when(c >= 1)
                    def _():
                        wb_wait(nb)

                    @pl.when(c + 1 < n_ch)
                    def _():
                        fire(nb)

                    drain(b)

                    @pl.loop(0, R)
                    def _(j):
                        _zero_padding_rows(idxb[b].at[j], outb[b].at[j])

                    @pl.when(c + 2 < n_ch)
                    def _():
                        idx_req(c + 2, b)

                    wb_start(c, b)

            wb_wait((n_ch - 1) % 2)

        run(src_table_hbm, src_tok_hbm, src_out_hbm)
        run(tgt_table_hbm, tgt_tok_hbm, tgt_out_hbm)

    return emb_kernel


def kernel(src_tokens, tgt_tokens, src_table, tgt_table):
    b, l = src_tokens.shape
    src_idx = src_tokens.astype(jnp.int32)
    tgt_idx = tgt_tokens.astype(jnp.int32)
    src_lin = _tc_relayout_table(src_table).reshape(-1, 16)
    tgt_lin = _tc_relayout_table(tgt_table).reshape(-1, 16)
    emb = _make_kernel(b, l)
    src_out, tgt_out = emb(src_lin, tgt_lin, src_idx, tgt_idx)
    return (src_out.reshape(b, l, DIM), tgt_out.reshape(b, l, DIM))


# speedup vs baseline: 2.2963x; 2.2963x over previous
import dataclasses
import functools

import jax
import jax.numpy as jnp
from jax import lax
from jax.experimental import pallas as pl
from jax.experimental.pallas import tpu as pltpu
from jax.experimental.pallas import tpu_sc as plsc

DIM = 32
R = 8
LANES = 16
NTEC = 32
SPLITS = ((0, 128), (128, 72))


def _zero_padding_rows(idx_row, out_row):
    zeros = jnp.zeros((LANES,), jnp.float32)
    for off in list(range(0, 192, LANES)) + [200 - LANES]:
        v = idx_row[pl.ds(off, LANES)]
        is_pad = v == 0

        @pl.when(jnp.any(is_pad))
        def _():
            rows = jnp.arange(LANES, dtype=jnp.int32) + off

            @pl.loop(0, DIM)
            def _(c):
                cols = jnp.full((LANES,), 0, jnp.int32) + c
                plsc.store_scatter(out_row, [rows, cols], zeros, mask=is_pad)


def _make_kernel(n_rows, n_cols):
    mesh = plsc.VectorSubcoreMesh(core_axis_name="c", subcore_axis_name="s")
    out_sds = jax.ShapeDtypeStruct((n_rows, n_cols, DIM), jnp.float32)
    rows_per_tec = n_rows // NTEC
    n_ch = rows_per_tec // R

    cp = pltpu.CompilerParams()
    fields = pltpu.CompilerParams.__dataclass_fields__
    if "needs_layout_passes" in fields:
        cp = dataclasses.replace(cp, needs_layout_passes=False)
    if "use_tc_tiling_on_sc" in fields:
        cp = dataclasses.replace(cp, use_tc_tiling_on_sc=False)

    @functools.partial(
        pl.kernel,
        out_type=out_sds,
        mesh=mesh,
        compiler_params=cp,
        scratch_types=[
            pltpu.VMEM((R, n_cols), jnp.int32),
            pltpu.VMEM((R, n_cols), jnp.int32),
            pltpu.VMEM((R, n_cols, DIM), jnp.float32),
            pltpu.VMEM((R, n_cols, DIM), jnp.float32),
            pltpu.SemaphoreType.DMA,
            pltpu.SemaphoreType.DMA,
            pltpu.SemaphoreType.DMA,
            pltpu.SemaphoreType.DMA,
            pltpu.SemaphoreType.DMA,
            pltpu.SemaphoreType.DMA,
        ],
    )
    def emb_kernel(table_hbm, tok_hbm, out_hbm,
                   idx0, idx1, outb0, outb1, si0, si1, sg0, sg1, so0, so1):
        wid = lax.axis_index("s") * 2 + lax.axis_index("c")
        base_row = wid * rows_per_tec
        idxb = (idx0, idx1)
        outb = (outb0, outb1)
        si = (si0, si1)
        sg = (sg0, sg1)
        so = (so0, so1)

        def run(tbl, tok, out):
            def row0(c):
                return base_row + c * R

            def idx_req(c, b):
                pltpu.async_copy(tok.at[pl.ds(row0(c), R)], idxb[b], si[b])

            def idx_wait(b):
                pltpu.make_async_copy(
                    tok.at[pl.ds(0, R)], idxb[b], si[b]).wait()

            def fire(b):
                @pl.loop(0, R)
                def _(j):
                    for off, w in SPLITS:
                        pltpu.async_copy(
                            tbl.at[idxb[b].at[j, pl.ds(off, w)]],
                            outb[b].at[j, pl.ds(off, w)], sg[b])

            def drain(b):
                @pl.loop(0, R)
                def _(j):
                    for off, w in SPLITS:
                        pltpu.make_async_copy(
                            tbl.at[idxb[b].at[j, pl.ds(off, w)]],
                            outb[b].at[j, pl.ds(off, w)], sg[b]).wait()

            def wb_start(c, b):
                pltpu.async_copy(outb[b], out.at[pl.ds(row0(c), R)], so[b])

            def wb_wait(b):
                pltpu.make_async_copy(
                    outb[b], out.at[pl.ds(0, R)], so[b]).wait()

            idx_req(0, 0)
            idx_req(1, 1)
            idx_wait(0)
            fire(0)

            @pl.loop(0, n_ch // 2)
            def _(k):
                for half in (0, 1):
                    c = 2 * k + half
                    b = half
                    nb = 1 - half
                    @pl.when(c + 1 < n_ch)
                    def _():
                        idx_wait(nb)

                    @pl.when(c >= 1)
                    def _():
                        wb_wait(nb)

                    @pl.when(c + 1 < n_ch)
                    def _():
                        fire(nb)

                    drain(b)

                    @pl.loop(0, R)
                    def _(j):
                        _zero_padding_rows(idxb[b].at[j], outb[b].at[j])

                    @pl.when(c + 2 < n_ch)
                    def _():
                        idx_req(c + 2, b)

                    wb_start(c, b)

            wb_wait((n_ch - 1) % 2)

        run(table_hbm, tok_hbm, out_hbm)

    return emb_kernel


def kernel(src_tokens, tgt_tokens, src_table, tgt_table):
    b, l = src_tokens.shape
    src_idx = src_tokens.astype(jnp.int32)
    tgt_idx = tgt_tokens.astype(jnp.int32)
    emb = _make_kernel(b, l)
    return (emb(src_table, src_idx), emb(tgt_table, tgt_idx))
